# baseline (speedup 1.0000x reference)
import functools

import jax
import jax.numpy as jnp
from jax import lax
from jax.experimental import pallas as pl
from jax.experimental.pallas import tpu as pltpu
from jax.experimental.pallas import tpu_sc as plsc

N = 10000
E = 320000
D = 128
H = 64
G = 64
NP = 10240
W1 = 80
EPS = 1e-5

NC = 2
NS = 16
ROWS_PER_TILE = NP // NS
NACC = 10016
RPT_ACC = NACC // NS

EPT = E // NS
CH = 160


def _sc_mesh():
    return plsc.VectorSubcoreMesh(core_axis_name="c", subcore_axis_name="s")


def _chunk_loop(tab_h, src_v, dst_v, acc_sh, bufs, gs, ss, extra=None):
    b0, b1 = bufs[0], bufs[1]
    sem0, sem1 = gs[0], gs[1]
    pltpu.async_copy(tab_h.at[src_v.at[0]], b0, sem0)

    def step(j, b, sem, b_n, sem_n, nxt):
        if nxt is not None:
            pltpu.async_copy(tab_h.at[src_v.at[nxt]], b_n, sem_n)
        pltpu.make_async_copy(tab_h.at[src_v.at[j]], b, sem).wait()
        pltpu.sync_copy(b, acc_sh.at[dst_v.at[j]], add=True)
        if extra is not None:
            extra(j)

    def body(i, carry):
        j0 = 2 * i
        step(j0, b0, sem0, b1, sem1, j0 + 1)

        @pl.when(i < CH // 2 - 1)
        def _():
            pltpu.async_copy(tab_h.at[src_v.at[j0 + 2]], b0, sem0)

        pltpu.make_async_copy(tab_h.at[src_v.at[j0 + 1]], b1, sem1).wait()
        pltpu.sync_copy(b1, acc_sh.at[dst_v.at[j0 + 1]], add=True)
        if extra is not None:
            extra(j0 + 1)
        return carry

    lax.fori_loop(0, CH // 2, body, 0, unroll=False)


def _agg_one(table2, srcL3, srcR3, dst3, zeros, with_deg):
    out_type = [jax.ShapeDtypeStruct((NP, H), jnp.float32)] * 2
    scratch = [
        pltpu.VMEM((CH, 128), jnp.int32),
        pltpu.VMEM((CH, 128), jnp.int32),
    ] + [pltpu.VMEM((128, H), jnp.float32)] * 4 + [
        pltpu.VMEM_SHARED((NACC, H), jnp.float32),
    ] + [pltpu.SemaphoreType.DMA] * 8
    extra_in = ()
    if with_deg:
        out_type = out_type + [jax.ShapeDtypeStruct((NP, 8), jnp.float32)]
        scratch = scratch + [pltpu.VMEM((128, 8), jnp.float32),
                             pltpu.VMEM_SHARED((NACC, 8), jnp.float32),
                             pltpu.SemaphoreType.DMA]
        extra_in = (jnp.ones((128, 8), jnp.float32),
                    jnp.zeros((NP, 8), jnp.float32))

    @functools.partial(
        pl.kernel,
        out_type=out_type,
        mesh=_sc_mesh(),
        compiler_params=pltpu.CompilerParams(use_tc_tiling_on_sc=False),
        scratch_types=scratch,
    )
    def k(tab_h, srcL_h, srcR_h, dst_h, zero_h, *rest):
        if with_deg:
            ones_h, z8_h, outL_h, outR_h, deg_h = rest[:5]
            rest = rest[5:]
        else:
            outL_h, outR_h = rest[:2]
            rest = rest[2:]
        src_v, dst_v, b0, b1, b2, b3, acc_sh = rest[:7]
        gs = rest[7:11]
        ss = rest[11:15]
        if with_deg:
            ones_v, dacc_sh, sd = rest[15:18]
        bufs = (b0, b1, b2, b3)
        cid = lax.axis_index("c")
        sid = lax.axis_index("s")
        r0 = sid * RPT_ACC
        pltpu.sync_copy(zero_h.at[pl.ds(r0, RPT_ACC)],
                        acc_sh.at[pl.ds(r0, RPT_ACC)])

        @pl.when(cid == 0)
        def _():
            pltpu.sync_copy(srcL_h.at[sid], src_v)

        @pl.when(cid == 1)
        def _():
            pltpu.sync_copy(srcR_h.at[sid], src_v)

        pltpu.sync_copy(dst_h.at[sid], dst_v)

        extra = None
        if with_deg:
            @pl.when(cid == 0)
            def _():
                pltpu.sync_copy(ones_h, ones_v)
                pltpu.sync_copy(z8_h.at[pl.ds(r0, RPT_ACC)],
                                dacc_sh.at[pl.ds(r0, RPT_ACC)])

            def extra(j):
                @pl.when(cid == 0)
                def _():
                    pltpu.async_copy(ones_v, dacc_sh.at[dst_v.at[j]], sd,
                                     add=True)

                    @pl.when(j >= 2)
                    def _():
                        pltpu.make_async_copy(
                            ones_v, dacc_sh.at[dst_v.at[0]], sd).wait()

        plsc.subcore_barrier()
        _chunk_loop(tab_h, src_v, dst_v, acc_sh, bufs, gs, ss, extra=extra)

        if with_deg:
            @pl.when(cid == 0)
            def _():
                for _i in range(2):
                    pltpu.make_async_copy(
                        ones_v, dacc_sh.at[dst_v.at[0]], sd).wait()
                pltpu.sync_copy(dacc_sh.at[pl.ds(r0, RPT_ACC)],
                                deg_h.at[pl.ds(r0, RPT_ACC)])

        plsc.subcore_barrier()

        @pl.when(cid == 0)
        def _():
            pltpu.sync_copy(acc_sh.at[pl.ds(r0, RPT_ACC)],
                            outL_h.at[pl.ds(r0, RPT_ACC)])

        @pl.when(cid == 1)
        def _():
            pltpu.sync_copy(acc_sh.at[pl.ds(r0, RPT_ACC)],
                            outR_h.at[pl.ds(r0, RPT_ACC)])

    return k(table2, srcL3, srcR3, dst3, zeros, *extra_in)


def _agg_two(tab2a, tab2b, srcL3, srcR3, dst3, zeros):
    @functools.partial(
        pl.kernel,
        out_type=[jax.ShapeDtypeStruct((NP, H), jnp.float32)] * 4,
        mesh=_sc_mesh(),
        compiler_params=pltpu.CompilerParams(use_tc_tiling_on_sc=False),
        scratch_types=[
            pltpu.VMEM((CH, 128), jnp.int32),
            pltpu.VMEM((CH, 128), jnp.int32),
        ] + [pltpu.VMEM((128, H), jnp.float32)] * 4 + [
            pltpu.VMEM_SHARED((NACC, H), jnp.float32),
        ] + [pltpu.SemaphoreType.DMA] * 8,
    )
    def k(ta_h, tb_h, srcL_h, srcR_h, dst_h, zero_h,
          outaL_h, outaR_h, outbL_h, outbR_h,
          src_v, dst_v, b0, b1, b2, b3, acc_sh,
          g0, g1, g2, g3, s0, s1, s2, s3):
        bufs = (b0, b1, b2, b3)
        gs = (g0, g1, g2, g3)
        ss = (s0, s1, s2, s3)
        cid = lax.axis_index("c")
        sid = lax.axis_index("s")
        r0 = sid * RPT_ACC

        @pl.when(cid == 0)
        def _():
            pltpu.sync_copy(srcL_h.at[sid], src_v)

        @pl.when(cid == 1)
        def _():
            pltpu.sync_copy(srcR_h.at[sid], src_v)

        pltpu.sync_copy(dst_h.at[sid], dst_v)

        pltpu.sync_copy(zero_h.at[pl.ds(r0, RPT_ACC)],
                        acc_sh.at[pl.ds(r0, RPT_ACC)])
        plsc.subcore_barrier()
        _chunk_loop(ta_h, src_v, dst_v, acc_sh, bufs, gs, ss)
        plsc.subcore_barrier()

        @pl.when(cid == 0)
        def _():
            pltpu.sync_copy(acc_sh.at[pl.ds(r0, RPT_ACC)],
                            outaL_h.at[pl.ds(r0, RPT_ACC)])

        @pl.when(cid == 1)
        def _():
            pltpu.sync_copy(acc_sh.at[pl.ds(r0, RPT_ACC)],
                            outaR_h.at[pl.ds(r0, RPT_ACC)])

        plsc.subcore_barrier()
        pltpu.sync_copy(zero_h.at[pl.ds(r0, RPT_ACC)],
                        acc_sh.at[pl.ds(r0, RPT_ACC)])
        plsc.subcore_barrier()
        _chunk_loop(tb_h, src_v, dst_v, acc_sh, bufs, gs, ss)
        plsc.subcore_barrier()

        @pl.when(cid == 0)
        def _():
            pltpu.sync_copy(acc_sh.at[pl.ds(r0, RPT_ACC)],
                            outbL_h.at[pl.ds(r0, RPT_ACC)])

        @pl.when(cid == 1)
        def _():
            pltpu.sync_copy(acc_sh.at[pl.ds(r0, RPT_ACC)],
                            outbR_h.at[pl.ds(r0, RPT_ACC)])

    return k(tab2a, tab2b, srcL3, srcR3, dst3, zeros)


_BLK = 1024
_GRID = NP // _BLK


def _ln(h, g, b):
    mu = jnp.mean(h, axis=-1, keepdims=True)
    v = jnp.mean((h - mu) * (h - mu), axis=-1, keepdims=True)
    return (h - mu) * lax.rsqrt(v + EPS) * g + b


def _row_spec(w):
    return pl.BlockSpec((_BLK, w), lambda i: (i, 0))


def _full_spec(shape):
    nd = len(shape)
    return pl.BlockSpec(shape, lambda i: (0,) * nd)


def _tc_round1(xp, accL, accR, degp, wlin, ws1, wn1, bpack, lnpack):
    def body(x_r, al_r, ar_r, dp_r, wl_r, ws_r, wn_r, bp_r, ln_r,
             out1_r, z2_r, z3_r, inv_r):
        al = al_r[...]
        ar = ar_r[...]
        deg = dp_r[...][:, 0:1]
        pos = deg > 0.0
        inv = jnp.where(pos, 1.0 / jnp.maximum(deg, 1.0), 0.0)
        mcol = jnp.where(pos, 1.0, 0.0)
        accx = jnp.concatenate([al, ar], axis=1)
        ax = jnp.where(pos, accx * inv, 0.0)
        xb = x_r[...]
        wl = wl_r[...]
        ws = ws_r[...]
        wn = wn_r[...]
        bp = bp_r[...]
        lnp = ln_r[...]
        outs = []
        for l in range(3):
            linb = bp[l:l + 1, :]
            convb = bp[3 + l:4 + l, :]
            h = jnp.dot(xb, wl[l], preferred_element_type=jnp.float32) + linb
            a = jnp.dot(ax, wl[l], preferred_element_type=jnp.float32) + mcol * linb
            pre = (jnp.dot(h, ws[l], preferred_element_type=jnp.float32)
                   + jnp.dot(a, wn[l], preferred_element_type=jnp.float32) + convb)
            outs.append(pre)
        out1_r[...] = outs[0]
        z2_r[...] = jnp.maximum(_ln(outs[1], lnp[0:1, :], lnp[1:2, :]), 0.0)
        z3_r[...] = jnp.maximum(_ln(outs[2], lnp[2:3, :], lnp[3:4, :]), 0.0)
        inv_r[...] = jnp.broadcast_to(inv, (_BLK, D))

    out_shapes = [jax.ShapeDtypeStruct((NP, D), jnp.float32)] * 4
    return pl.pallas_call(
        body,
        grid=(_GRID,),
        in_specs=[
            _row_spec(D), _row_spec(H), _row_spec(H),
            pl.BlockSpec((_BLK, 8), lambda i: (i, 0)),
            _full_spec((3, D, D)), _full_spec((3, D, D)), _full_spec((3, D, D)),
            _full_spec((8, D)), _full_spec((8, D)),
        ],
        out_specs=[_row_spec(D)] * 4,
        out_shape=out_shapes,
    )(xp, accL, accR, degp, wlin, ws1, wn1, bpack, lnpack)


def _tc_round2(z2, z3a, a2L, a2R, a3L, a3R, inv, ws2, wn2, bp2):
    def body(z2_r, z3_r, a2l_r, a2r_r, a3l_r, a3r_r, inv_r, ws_r, wn_r, bp_r,
             out2_r, z3b_r):
        inv = inv_r[...][:, 0:1]
        pos = inv > 0.0
        ws = ws_r[...]
        wn = wn_r[...]
        bp = bp_r[...]
        a2 = jnp.concatenate([a2l_r[...], a2r_r[...]], axis=1)
        a3 = jnp.concatenate([a3l_r[...], a3r_r[...]], axis=1)
        a2 = jnp.where(pos, a2 * inv, 0.0)
        a3 = jnp.where(pos, a3 * inv, 0.0)
        out2_r[...] = (jnp.dot(z2_r[...], ws[0], preferred_element_type=jnp.float32)
                       + jnp.dot(a2, wn[0], preferred_element_type=jnp.float32)
                       + bp[0:1, :])
        pre = (jnp.dot(z3_r[...], ws[1], preferred_element_type=jnp.float32)
               + jnp.dot(a3, wn[1], preferred_element_type=jnp.float32)
               + bp[1:2, :])
        z3b_r[...] = jnp.maximum(_ln(pre, bp[2:3, :], bp[3:4, :]), 0.0)

    out_shapes = [jax.ShapeDtypeStruct((NP, D), jnp.float32)] * 2
    return pl.pallas_call(
        body,
        grid=(_GRID,),
        in_specs=[
            _row_spec(D), _row_spec(D),
            _row_spec(H), _row_spec(H), _row_spec(H), _row_spec(H),
            _row_spec(D),
            _full_spec((2, D, D)), _full_spec((2, D, D)), _full_spec((8, D)),
        ],
        out_specs=[_row_spec(D)] * 2,
        out_shape=out_shapes,
    )(z2, z3a, a2L, a2R, a3L, a3R, inv, ws2, wn2, bp2)


def _tc_round3(out1, out2, z3b, a4L, a4R, inv, batch2d, ws3, wn3, wf, fp):
    def body(o1_r, o2_r, z3_r, a4l_r, a4r_r, inv_r, b_r, ws_r, wn_r, wf_r, fp_r,
             node_r, graph_r, gsum):
        i = pl.program_id(0)
        inv = inv_r[...][:, 0:1]
        pos = inv > 0.0
        fp = fp_r[...]
        wf = wf_r[...]
        a4 = jnp.concatenate([a4l_r[...], a4r_r[...]], axis=1)
        a4 = jnp.where(pos, a4 * inv, 0.0)
        out3 = (jnp.dot(z3_r[...], ws_r[...], preferred_element_type=jnp.float32)
                + jnp.dot(a4, wn_r[...], preferred_element_type=jnp.float32)
                + fp[3:4, :])
        xf = o1_r[...] + o2_r[...] + out3
        node_r[...] = jnp.maximum(
            _ln(jnp.dot(xf, wf, preferred_element_type=jnp.float32) + fp[0:1, :],
                fp[1:2, :], fp[2:3, :]), 0.0)

        @pl.when(i == 0)
        def _():
            gsum[...] = jnp.zeros((G, D), jnp.float32)

        ids = b_r[...]
        iota = lax.broadcasted_iota(jnp.int32, (_BLK, G), 1)
        oh = (ids == iota).astype(jnp.float32)
        gsum[...] += lax.dot_general(oh, xf, (((0,), (0,)), ((), ())),
                                     preferred_element_type=jnp.float32)

        @pl.when(i == _GRID - 1)
        def _():
            gs = gsum[...]
            graph_r[...] = jnp.maximum(
                _ln(jnp.dot(gs, wf, preferred_element_type=jnp.float32) + fp[0:1, :],
                    fp[1:2, :], fp[2:3, :]), 0.0)

    out_shapes = [jax.ShapeDtypeStruct((NP, D), jnp.float32),
                  jax.ShapeDtypeStruct((G, D), jnp.float32)]
    return pl.pallas_call(
        body,
        grid=(_GRID,),
        in_specs=[
            _row_spec(D), _row_spec(D), _row_spec(D),
            _row_spec(H), _row_spec(H),
            _row_spec(D),
            pl.BlockSpec((_BLK, 1), lambda i: (i, 0)),
            _full_spec((D, D)), _full_spec((D, D)), _full_spec((D, D)),
            _full_spec((8, D)),
        ],
        out_specs=[_row_spec(D), _full_spec((G, D))],
        out_shape=out_shapes,
        scratch_shapes=[pltpu.VMEM((G, D), jnp.float32)],
    )(out1, out2, z3b, a4L, a4R, inv, batch2d, ws3, wn3, wf, fp)


def _edge_layout(idx, fill):
    a = idx.reshape(NS, EPT)
    a = jnp.pad(a, ((0, 0), (0, CH * 128 - EPT)), constant_values=fill)
    return a.reshape(NS, CH, 128)


def kernel(x, edge_index, batch, params):
    src = edge_index[0].astype(jnp.int32)
    dst = edge_index[1].astype(jnp.int32)

    xp = jnp.pad(x, ((0, NP - N), (0, 0)))

    srcL = _edge_layout(src * 2, 0)
    srcR = _edge_layout(src * 2 + 1, 1)
    dstB = _edge_layout(dst, N)

    z64 = jnp.zeros((NP, H), jnp.float32)

    batch2d = jnp.pad(batch.astype(jnp.int32), (0, NP - N),
                      constant_values=G).reshape(NP, 1)

    lv = params["levels"]
    f = params["fusion"]
    zrow = jnp.zeros((D,), jnp.float32)
    wlin = jnp.stack([l["lin_W"] for l in lv])
    ws1 = jnp.stack([l["convs"][0]["Ws"] for l in lv])
    wn1 = jnp.stack([l["convs"][0]["Wn"] for l in lv])
    bpack = jnp.stack(
        [lv[0]["lin_b"], lv[1]["lin_b"], lv[2]["lin_b"],
         lv[0]["convs"][0]["b"], lv[1]["convs"][0]["b"], lv[2]["convs"][0]["b"],
         zrow, zrow])
    lnpack = jnp.stack(
        [lv[1]["lns"][0]["g"], lv[1]["lns"][0]["b"],
         lv[2]["lns"][0]["g"], lv[2]["lns"][0]["b"], zrow, zrow, zrow, zrow])
    ws2 = jnp.stack([lv[1]["convs"][1]["Ws"], lv[2]["convs"][1]["Ws"]])
    wn2 = jnp.stack([lv[1]["convs"][1]["Wn"], lv[2]["convs"][1]["Wn"]])
    bp2 = jnp.stack(
        [lv[1]["convs"][1]["b"], lv[2]["convs"][1]["b"],
         lv[2]["lns"][1]["g"], lv[2]["lns"][1]["b"], zrow, zrow, zrow, zrow])
    ws3 = lv[2]["convs"][2]["Ws"]
    wn3 = lv[2]["convs"][2]["Wn"]
    fp = jnp.stack(
        [f["b"], f["ln_g"], f["ln_b"], lv[2]["convs"][2]["b"],
         zrow, zrow, zrow, zrow])

    accL, accR, degp = _agg_one(xp.reshape(NP * 2, H), srcL, srcR, dstB, z64,
                                with_deg=True)

    out1, z2, z3a, inv = _tc_round1(xp, accL, accR, degp,
                                    wlin, ws1, wn1, bpack, lnpack)

    a2L, a2R, a3L, a3R = _agg_two(z2.reshape(NP * 2, H), z3a.reshape(NP * 2, H),
                                  srcL, srcR, dstB, z64)

    out2, z3b = _tc_round2(z2, z3a, a2L, a2R, a3L, a3R, inv, ws2, wn2, bp2)

    a4L, a4R = _agg_one(z3b.reshape(NP * 2, H), srcL, srcR, dstB, z64,
                        with_deg=False)

    node_p, graph_out = _tc_round3(out1, out2, z3b, a4L, a4R, inv,
                                   batch2d, ws3, wn3, f["W"], fp)

    return (node_p[:N], graph_out)


# speedup vs baseline: 1.0001x; 1.0001x over previous
import functools

import jax
import jax.numpy as jnp
from jax import lax
from jax.experimental import pallas as pl
from jax.experimental.pallas import tpu as pltpu
from jax.experimental.pallas import tpu_sc as plsc

N = 10000
E = 320000
D = 128
H = 64
G = 64
NP = 10240
W1 = 80
EPS = 1e-5

NC = 2
NS = 16
ROWS_PER_TILE = NP // NS
NACC = 10240
RPT_ACC = NACC // NS

EPT = E // NS
CH = 160


def _sc_mesh():
    return plsc.VectorSubcoreMesh(core_axis_name="c", subcore_axis_name="s")


def _chunk_loop(tab_h, src_v, dst_v, acc_sh, bufs, gs, ss, extra=None):
    b0, b1 = bufs[0], bufs[1]
    sem0, sem1 = gs[0], gs[1]
    pltpu.async_copy(tab_h.at[src_v.at[0]], b0, sem0)

    def step(j, b, sem, b_n, sem_n, nxt):
        if nxt is not None:
            pltpu.async_copy(tab_h.at[src_v.at[nxt]], b_n, sem_n)
        pltpu.make_async_copy(tab_h.at[src_v.at[j]], b, sem).wait()
        pltpu.sync_copy(b, acc_sh.at[dst_v.at[j]], add=True)
        if extra is not None:
            extra(j)

    def body(i, carry):
        j0 = 2 * i
        step(j0, b0, sem0, b1, sem1, j0 + 1)

        @pl.when(i < CH // 2 - 1)
        def _():
            pltpu.async_copy(tab_h.at[src_v.at[j0 + 2]], b0, sem0)

        pltpu.make_async_copy(tab_h.at[src_v.at[j0 + 1]], b1, sem1).wait()
        pltpu.sync_copy(b1, acc_sh.at[dst_v.at[j0 + 1]], add=True)
        if extra is not None:
            extra(j0 + 1)
        return carry

    lax.fori_loop(0, CH // 2, body, 0, unroll=False)


def _agg_one(table2, srcL3, srcR3, dst3, zeros, with_deg):
    out_type = [jax.ShapeDtypeStruct((NP, H), jnp.float32)] * 2
    scratch = [
        pltpu.VMEM((CH, 128), jnp.int32),
        pltpu.VMEM((CH, 128), jnp.int32),
    ] + [pltpu.VMEM((128, H), jnp.float32)] * 4 + [
        pltpu.VMEM_SHARED((NACC, H), jnp.float32),
    ] + [pltpu.SemaphoreType.DMA] * 8
    extra_in = ()
    if with_deg:
        out_type = out_type + [jax.ShapeDtypeStruct((NP, 8), jnp.float32)]
        scratch = scratch + [pltpu.VMEM((128, 8), jnp.float32),
                             pltpu.VMEM_SHARED((NACC, 8), jnp.float32),
                             pltpu.SemaphoreType.DMA]
        extra_in = (jnp.ones((128, 8), jnp.float32),
                    jnp.zeros((NP, 8), jnp.float32))

    @functools.partial(
        pl.kernel,
        out_type=out_type,
        mesh=_sc_mesh(),
        compiler_params=pltpu.CompilerParams(use_tc_tiling_on_sc=False),
        scratch_types=scratch,
    )
    def k(tab_h, srcL_h, srcR_h, dst_h, zero_h, *rest):
        if with_deg:
            ones_h, z8_h, outL_h, outR_h, deg_h = rest[:5]
            rest = rest[5:]
        else:
            outL_h, outR_h = rest[:2]
            rest = rest[2:]
        src_v, dst_v, b0, b1, b2, b3, acc_sh = rest[:7]
        gs = rest[7:11]
        ss = rest[11:15]
        if with_deg:
            ones_v, dacc_sh, sd = rest[15:18]
        bufs = (b0, b1, b2, b3)
        cid = lax.axis_index("c")
        sid = lax.axis_index("s")
        r0 = sid * RPT_ACC
        pltpu.sync_copy(zero_h.at[pl.ds(r0, RPT_ACC)],
                        acc_sh.at[pl.ds(r0, RPT_ACC)])

        @pl.when(cid == 0)
        def _():
            pltpu.sync_copy(srcL_h.at[sid], src_v)

        @pl.when(cid == 1)
        def _():
            pltpu.sync_copy(srcR_h.at[sid], src_v)

        pltpu.sync_copy(dst_h.at[sid], dst_v)

        extra = None
        if with_deg:
            @pl.when(cid == 0)
            def _():
                pltpu.sync_copy(ones_h, ones_v)
                pltpu.sync_copy(z8_h.at[pl.ds(r0, RPT_ACC)],
                                dacc_sh.at[pl.ds(r0, RPT_ACC)])

            def extra(j):
                @pl.when(cid == 0)
                def _():
                    pltpu.async_copy(ones_v, dacc_sh.at[dst_v.at[j]], sd,
                                     add=True)

                    @pl.when(j >= 2)
                    def _():
                        pltpu.make_async_copy(
                            ones_v, dacc_sh.at[dst_v.at[0]], sd).wait()

        plsc.subcore_barrier()
        _chunk_loop(tab_h, src_v, dst_v, acc_sh, bufs, gs, ss, extra=extra)

        if with_deg:
            @pl.when(cid == 0)
            def _():
                for _i in range(2):
                    pltpu.make_async_copy(
                        ones_v, dacc_sh.at[dst_v.at[0]], sd).wait()
                pltpu.sync_copy(dacc_sh.at[pl.ds(r0, RPT_ACC)],
                                deg_h.at[pl.ds(r0, RPT_ACC)])

        plsc.subcore_barrier()

        @pl.when(cid == 0)
        def _():
            pltpu.sync_copy(acc_sh.at[pl.ds(r0, RPT_ACC)],
                            outL_h.at[pl.ds(r0, RPT_ACC)])

        @pl.when(cid == 1)
        def _():
            pltpu.sync_copy(acc_sh.at[pl.ds(r0, RPT_ACC)],
                            outR_h.at[pl.ds(r0, RPT_ACC)])

    return k(table2, srcL3, srcR3, dst3, zeros, *extra_in)


def _agg_two(tab2a, tab2b, srcL3, srcR3, dst3, zeros):
    @functools.partial(
        pl.kernel,
        out_type=[jax.ShapeDtypeStruct((NP, H), jnp.float32)] * 4,
        mesh=_sc_mesh(),
        compiler_params=pltpu.CompilerParams(use_tc_tiling_on_sc=False),
        scratch_types=[
            pltpu.VMEM((CH, 128), jnp.int32),
            pltpu.VMEM((CH, 128), jnp.int32),
        ] + [pltpu.VMEM((128, H), jnp.float32)] * 4 + [
            pltpu.VMEM_SHARED((NACC, H), jnp.float32),
        ] + [pltpu.SemaphoreType.DMA] * 8,
    )
    def k(ta_h, tb_h, srcL_h, srcR_h, dst_h, zero_h,
          outaL_h, outaR_h, outbL_h, outbR_h,
          src_v, dst_v, b0, b1, b2, b3, acc_sh,
          g0, g1, g2, g3, s0, s1, s2, s3):
        bufs = (b0, b1, b2, b3)
        gs = (g0, g1, g2, g3)
        ss = (s0, s1, s2, s3)
        cid = lax.axis_index("c")
        sid = lax.axis_index("s")
        r0 = sid * RPT_ACC

        @pl.when(cid == 0)
        def _():
            pltpu.sync_copy(srcL_h.at[sid], src_v)

        @pl.when(cid == 1)
        def _():
            pltpu.sync_copy(srcR_h.at[sid], src_v)

        pltpu.sync_copy(dst_h.at[sid], dst_v)

        pltpu.sync_copy(zero_h.at[pl.ds(r0, RPT_ACC)],
                        acc_sh.at[pl.ds(r0, RPT_ACC)])
        plsc.subcore_barrier()
        _chunk_loop(ta_h, src_v, dst_v, acc_sh, bufs, gs, ss)
        plsc.subcore_barrier()

        @pl.when(cid == 0)
        def _():
            pltpu.sync_copy(acc_sh.at[pl.ds(r0, RPT_ACC)],
                            outaL_h.at[pl.ds(r0, RPT_ACC)])

        @pl.when(cid == 1)
        def _():
            pltpu.sync_copy(acc_sh.at[pl.ds(r0, RPT_ACC)],
                            outaR_h.at[pl.ds(r0, RPT_ACC)])

        plsc.subcore_barrier()
        pltpu.sync_copy(zero_h.at[pl.ds(r0, RPT_ACC)],
                        acc_sh.at[pl.ds(r0, RPT_ACC)])
        plsc.subcore_barrier()
        _chunk_loop(tb_h, src_v, dst_v, acc_sh, bufs, gs, ss)
        plsc.subcore_barrier()

        @pl.when(cid == 0)
        def _():
            pltpu.sync_copy(acc_sh.at[pl.ds(r0, RPT_ACC)],
                            outbL_h.at[pl.ds(r0, RPT_ACC)])

        @pl.when(cid == 1)
        def _():
            pltpu.sync_copy(acc_sh.at[pl.ds(r0, RPT_ACC)],
                            outbR_h.at[pl.ds(r0, RPT_ACC)])

    return k(tab2a, tab2b, srcL3, srcR3, dst3, zeros)


_BLK = 1024
_GRID = NP // _BLK


def _ln(h, g, b):
    mu = jnp.mean(h, axis=-1, keepdims=True)
    v = jnp.mean((h - mu) * (h - mu), axis=-1, keepdims=True)
    return (h - mu) * lax.rsqrt(v + EPS) * g + b


def _row_spec(w):
    return pl.BlockSpec((_BLK, w), lambda i: (i, 0))


def _full_spec(shape):
    nd = len(shape)
    return pl.BlockSpec(shape, lambda i: (0,) * nd)


def _tc_round1(xp, accL, accR, degp, wlin, ws1, wn1, bpack, lnpack):
    def body(x_r, al_r, ar_r, dp_r, wl_r, ws_r, wn_r, bp_r, ln_r,
             out1_r, z2_r, z3_r, inv_r):
        al = al_r[...]
        ar = ar_r[...]
        deg = dp_r[...][:, 0:1]
        pos = deg > 0.0
        inv = jnp.where(pos, 1.0 / jnp.maximum(deg, 1.0), 0.0)
        mcol = jnp.where(pos, 1.0, 0.0)
        accx = jnp.concatenate([al, ar], axis=1)
        ax = jnp.where(pos, accx * inv, 0.0)
        xb = x_r[...]
        wl = wl_r[...]
        ws = ws_r[...]
        wn = wn_r[...]
        bp = bp_r[...]
        lnp = ln_r[...]
        outs = []
        for l in range(3):
            linb = bp[l:l + 1, :]
            convb = bp[3 + l:4 + l, :]
            h = jnp.dot(xb, wl[l], preferred_element_type=jnp.float32) + linb
            a = jnp.dot(ax, wl[l], preferred_element_type=jnp.float32) + mcol * linb
            pre = (jnp.dot(h, ws[l], preferred_element_type=jnp.float32)
                   + jnp.dot(a, wn[l], preferred_element_type=jnp.float32) + convb)
            outs.append(pre)
        out1_r[...] = outs[0]
        z2_r[...] = jnp.maximum(_ln(outs[1], lnp[0:1, :], lnp[1:2, :]), 0.0)
        z3_r[...] = jnp.maximum(_ln(outs[2], lnp[2:3, :], lnp[3:4, :]), 0.0)
        inv_r[...] = jnp.broadcast_to(inv, (_BLK, D))

    out_shapes = [jax.ShapeDtypeStruct((NP, D), jnp.float32)] * 4
    return pl.pallas_call(
        body,
        grid=(_GRID,),
        in_specs=[
            _row_spec(D), _row_spec(H), _row_spec(H),
            pl.BlockSpec((_BLK, 8), lambda i: (i, 0)),
            _full_spec((3, D, D)), _full_spec((3, D, D)), _full_spec((3, D, D)),
            _full_spec((8, D)), _full_spec((8, D)),
        ],
        out_specs=[_row_spec(D)] * 4,
        out_shape=out_shapes,
    )(xp, accL, accR, degp, wlin, ws1, wn1, bpack, lnpack)


def _tc_round2(z2, z3a, a2L, a2R, a3L, a3R, inv, ws2, wn2, bp2):
    def body(z2_r, z3_r, a2l_r, a2r_r, a3l_r, a3r_r, inv_r, ws_r, wn_r, bp_r,
             out2_r, z3b_r):
        inv = inv_r[...][:, 0:1]
        pos = inv > 0.0
        ws = ws_r[...]
        wn = wn_r[...]
        bp = bp_r[...]
        a2 = jnp.concatenate([a2l_r[...], a2r_r[...]], axis=1)
        a3 = jnp.concatenate([a3l_r[...], a3r_r[...]], axis=1)
        a2 = jnp.where(pos, a2 * inv, 0.0)
        a3 = jnp.where(pos, a3 * inv, 0.0)
        out2_r[...] = (jnp.dot(z2_r[...], ws[0], preferred_element_type=jnp.float32)
                       + jnp.dot(a2, wn[0], preferred_element_type=jnp.float32)
                       + bp[0:1, :])
        pre = (jnp.dot(z3_r[...], ws[1], preferred_element_type=jnp.float32)
               + jnp.dot(a3, wn[1], preferred_element_type=jnp.float32)
               + bp[1:2, :])
        z3b_r[...] = jnp.maximum(_ln(pre, bp[2:3, :], bp[3:4, :]), 0.0)

    out_shapes = [jax.ShapeDtypeStruct((NP, D), jnp.float32)] * 2
    return pl.pallas_call(
        body,
        grid=(_GRID,),
        in_specs=[
            _row_spec(D), _row_spec(D),
            _row_spec(H), _row_spec(H), _row_spec(H), _row_spec(H),
            _row_spec(D),
            _full_spec((2, D, D)), _full_spec((2, D, D)), _full_spec((8, D)),
        ],
        out_specs=[_row_spec(D)] * 2,
        out_shape=out_shapes,
    )(z2, z3a, a2L, a2R, a3L, a3R, inv, ws2, wn2, bp2)


def _tc_round3(out1, out2, z3b, a4L, a4R, inv, batch2d, ws3, wn3, wf, fp):
    def body(o1_r, o2_r, z3_r, a4l_r, a4r_r, inv_r, b_r, ws_r, wn_r, wf_r, fp_r,
             node_r, graph_r, gsum):
        i = pl.program_id(0)
        inv = inv_r[...][:, 0:1]
        pos = inv > 0.0
        fp = fp_r[...]
        wf = wf_r[...]
        a4 = jnp.concatenate([a4l_r[...], a4r_r[...]], axis=1)
        a4 = jnp.where(pos, a4 * inv, 0.0)
        out3 = (jnp.dot(z3_r[...], ws_r[...], preferred_element_type=jnp.float32)
                + jnp.dot(a4, wn_r[...], preferred_element_type=jnp.float32)
                + fp[3:4, :])
        xf = o1_r[...] + o2_r[...] + out3
        node_r[...] = jnp.maximum(
            _ln(jnp.dot(xf, wf, preferred_element_type=jnp.float32) + fp[0:1, :],
                fp[1:2, :], fp[2:3, :]), 0.0)

        @pl.when(i == 0)
        def _():
            gsum[...] = jnp.zeros((G, D), jnp.float32)

        ids = b_r[...]
        iota = lax.broadcasted_iota(jnp.int32, (_BLK, G), 1)
        oh = (ids == iota).astype(jnp.float32)
        gsum[...] += lax.dot_general(oh, xf, (((0,), (0,)), ((), ())),
                                     preferred_element_type=jnp.float32)

        @pl.when(i == _GRID - 1)
        def _():
            gs = gsum[...]
            graph_r[...] = jnp.maximum(
                _ln(jnp.dot(gs, wf, preferred_element_type=jnp.float32) + fp[0:1, :],
                    fp[1:2, :], fp[2:3, :]), 0.0)

    out_shapes = [jax.ShapeDtypeStruct((NP, D), jnp.float32),
                  jax.ShapeDtypeStruct((G, D), jnp.float32)]
    return pl.pallas_call(
        body,
        grid=(_GRID,),
        in_specs=[
            _row_spec(D), _row_spec(D), _row_spec(D),
            _row_spec(H), _row_spec(H),
            _row_spec(D),
            pl.BlockSpec((_BLK, 1), lambda i: (i, 0)),
            _full_spec((D, D)), _full_spec((D, D)), _full_spec((D, D)),
            _full_spec((8, D)),
        ],
        out_specs=[_row_spec(D), _full_spec((G, D))],
        out_shape=out_shapes,
        scratch_shapes=[pltpu.VMEM((G, D), jnp.float32)],
    )(out1, out2, z3b, a4L, a4R, inv, batch2d, ws3, wn3, wf, fp)


def _edge_layout(idx, fill):
    a = idx.reshape(NS, EPT)
    a = jnp.pad(a, ((0, 0), (0, CH * 128 - EPT)), constant_values=fill)
    return a.reshape(NS, CH, 128)


def kernel(x, edge_index, batch, params):
    src = edge_index[0].astype(jnp.int32)
    dst = edge_index[1].astype(jnp.int32)

    xp = jnp.pad(x, ((0, NP - N), (0, 0)))

    srcL = _edge_layout(src * 2, 0)
    srcR = _edge_layout(src * 2 + 1, 1)
    dstB = _edge_layout(dst, N)

    z64 = jnp.zeros((NP, H), jnp.float32)

    batch2d = jnp.pad(batch.astype(jnp.int32), (0, NP - N),
                      constant_values=G).reshape(NP, 1)

    lv = params["levels"]
    f = params["fusion"]
    zrow = jnp.zeros((D,), jnp.float32)
    wlin = jnp.stack([l["lin_W"] for l in lv])
    ws1 = jnp.stack([l["convs"][0]["Ws"] for l in lv])
    wn1 = jnp.stack([l["convs"][0]["Wn"] for l in lv])
    bpack = jnp.stack(
        [lv[0]["lin_b"], lv[1]["lin_b"], lv[2]["lin_b"],
         lv[0]["convs"][0]["b"], lv[1]["convs"][0]["b"], lv[2]["convs"][0]["b"],
         zrow, zrow])
    lnpack = jnp.stack(
        [lv[1]["lns"][0]["g"], lv[1]["lns"][0]["b"],
         lv[2]["lns"][0]["g"], lv[2]["lns"][0]["b"], zrow, zrow, zrow, zrow])
    ws2 = jnp.stack([lv[1]["convs"][1]["Ws"], lv[2]["convs"][1]["Ws"]])
    wn2 = jnp.stack([lv[1]["convs"][1]["Wn"], lv[2]["convs"][1]["Wn"]])
    bp2 = jnp.stack(
        [lv[1]["convs"][1]["b"], lv[2]["convs"][1]["b"],
         lv[2]["lns"][1]["g"], lv[2]["lns"][1]["b"], zrow, zrow, zrow, zrow])
    ws3 = lv[2]["convs"][2]["Ws"]
    wn3 = lv[2]["convs"][2]["Wn"]
    fp = jnp.stack(
        [f["b"], f["ln_g"], f["ln_b"], lv[2]["convs"][2]["b"],
         zrow, zrow, zrow, zrow])

    accL, accR, degp = _agg_one(xp.reshape(NP * 2, H), srcL, srcR, dstB, z64,
                                with_deg=True)

    out1, z2, z3a, inv = _tc_round1(xp, accL, accR, degp,
                                    wlin, ws1, wn1, bpack, lnpack)

    a2L, a2R, a3L, a3R = _agg_two(z2.reshape(NP * 2, H), z3a.reshape(NP * 2, H),
                                  srcL, srcR, dstB, z64)

    out2, z3b = _tc_round2(z2, z3a, a2L, a2R, a3L, a3R, inv, ws2, wn2, bp2)

    a4L, a4R = _agg_one(z3b.reshape(NP * 2, H), srcL, srcR, dstB, z64,
                        with_deg=False)

    node_p, graph_out = _tc_round3(out1, out2, z3b, a4L, a4R, inv,
                                   batch2d, ws3, wn3, f["W"], fp)

    return (node_p[:N], graph_out)


# speedup vs baseline: 1.5394x; 1.5393x over previous
import functools

import jax
import jax.numpy as jnp
from jax import lax
from jax.experimental import pallas as pl
from jax.experimental.pallas import tpu as pltpu
from jax.experimental.pallas import tpu_sc as plsc

N = 10000
E = 320000
D = 128
H = 64
G = 64
NP = 10240
W1 = 80
EPS = 1e-5

NC = 2
NS = 16
ROWS_PER_TILE = NP // NS
NACC = 10240
RPT_ACC = NACC // NS

EPT = E // NS
CH = 158


def _sc_mesh():
    return plsc.VectorSubcoreMesh(core_axis_name="c", subcore_axis_name="s")


def _chunk_loop(tab_h, src_v, dst_v, acc_sh, bufs, gs, ss, extra=None):
    b0, b1 = bufs[0], bufs[1]
    sem0, sem1 = gs[0], gs[1]
    pltpu.async_copy(tab_h.at[src_v.at[0]], b0, sem0)

    def step(j, b, sem, b_n, sem_n, nxt):
        if nxt is not None:
            pltpu.async_copy(tab_h.at[src_v.at[nxt]], b_n, sem_n)
        pltpu.make_async_copy(tab_h.at[src_v.at[j]], b, sem).wait()
        pltpu.sync_copy(b, acc_sh.at[dst_v.at[j]], add=True)
        if extra is not None:
            extra(j)

    def body(i, carry):
        j0 = 2 * i
        step(j0, b0, sem0, b1, sem1, j0 + 1)

        @pl.when(i < CH // 2 - 1)
        def _():
            pltpu.async_copy(tab_h.at[src_v.at[j0 + 2]], b0, sem0)

        pltpu.make_async_copy(tab_h.at[src_v.at[j0 + 1]], b1, sem1).wait()
        pltpu.sync_copy(b1, acc_sh.at[dst_v.at[j0 + 1]], add=True)
        if extra is not None:
            extra(j0 + 1)
        return carry

    lax.fori_loop(0, CH // 2, body, 0, unroll=False)


def _agg_one(table2, srcL3, srcR3, dst3, zeros, with_deg):
    out_type = [jax.ShapeDtypeStruct((NP, H), jnp.float32)] * 2
    scratch = [
        pltpu.VMEM((CH, 128), jnp.int32),
        pltpu.VMEM((CH, 128), jnp.int32),
    ] + [pltpu.VMEM((128, H), jnp.float32)] * 2 + [
        pltpu.VMEM_SHARED((NACC, H), jnp.float32),
    ] + [pltpu.SemaphoreType.DMA] * 2
    extra_in = ()
    if with_deg:
        out_type = out_type + [jax.ShapeDtypeStruct((NP, 8), jnp.float32)]
        scratch = scratch + [pltpu.VMEM((128, 8), jnp.float32),
                             pltpu.VMEM_SHARED((NACC, 8), jnp.float32),
                             pltpu.SemaphoreType.DMA]
        extra_in = (jnp.ones((128, 8), jnp.float32),
                    jnp.zeros((NP, 8), jnp.float32))

    @functools.partial(
        pl.kernel,
        out_type=out_type,
        mesh=_sc_mesh(),
        compiler_params=pltpu.CompilerParams(use_tc_tiling_on_sc=False),
        scratch_types=scratch,
    )
    def k(tab_h, srcL_h, srcR_h, dst_h, zero_h, *rest):
        if with_deg:
            ones_h, z8_h, outL_h, outR_h, deg_h = rest[:5]
            rest = rest[5:]
        else:
            outL_h, outR_h = rest[:2]
            rest = rest[2:]
        src_v, dst_v, b0, b1, acc_sh = rest[:5]
        gs = rest[5:7]
        ss = ()
        if with_deg:
            ones_v, dacc_sh, sd = rest[7:10]
        bufs = (b0, b1)
        cid = lax.axis_index("c")
        sid = lax.axis_index("s")
        r0 = sid * RPT_ACC
        pltpu.sync_copy(zero_h.at[pl.ds(r0, RPT_ACC)],
                        acc_sh.at[pl.ds(r0, RPT_ACC)])

        @pl.when(cid == 0)
        def _():
            pltpu.sync_copy(srcL_h.at[sid], src_v)

        @pl.when(cid == 1)
        def _():
            pltpu.sync_copy(srcR_h.at[sid], src_v)

        pltpu.sync_copy(dst_h.at[sid], dst_v)

        extra = None
        if with_deg:
            @pl.when(cid == 0)
            def _():
                pltpu.sync_copy(ones_h, ones_v)
                pltpu.sync_copy(z8_h.at[pl.ds(r0, RPT_ACC)],
                                dacc_sh.at[pl.ds(r0, RPT_ACC)])

            def extra(j):
                @pl.when(cid == 0)
                def _():
                    pltpu.async_copy(ones_v, dacc_sh.at[dst_v.at[j]], sd,
                                     add=True)

                    @pl.when(j >= 2)
                    def _():
                        pltpu.make_async_copy(
                            ones_v, dacc_sh.at[dst_v.at[0]], sd).wait()

        plsc.subcore_barrier()
        _chunk_loop(tab_h, src_v, dst_v, acc_sh, bufs, gs, ss, extra=extra)

        if with_deg:
            @pl.when(cid == 0)
            def _():
                for _i in range(2):
                    pltpu.make_async_copy(
                        ones_v, dacc_sh.at[dst_v.at[0]], sd).wait()
                pltpu.sync_copy(dacc_sh.at[pl.ds(r0, RPT_ACC)],
                                deg_h.at[pl.ds(r0, RPT_ACC)])

        plsc.subcore_barrier()

        @pl.when(cid == 0)
        def _():
            pltpu.sync_copy(acc_sh.at[pl.ds(r0, RPT_ACC)],
                            outL_h.at[pl.ds(r0, RPT_ACC)])

        @pl.when(cid == 1)
        def _():
            pltpu.sync_copy(acc_sh.at[pl.ds(r0, RPT_ACC)],
                            outR_h.at[pl.ds(r0, RPT_ACC)])

    return k(table2, srcL3, srcR3, dst3, zeros, *extra_in)


def _agg_two(tab2a, tab2b, srcL3, srcR3, dst3, zeros):
    @functools.partial(
        pl.kernel,
        out_type=[jax.ShapeDtypeStruct((NP, H), jnp.float32)] * 4,
        mesh=_sc_mesh(),
        compiler_params=pltpu.CompilerParams(use_tc_tiling_on_sc=False),
        scratch_types=[
            pltpu.VMEM((CH, 128), jnp.int32),
            pltpu.VMEM((CH, 128), jnp.int32),
        ] + [pltpu.VMEM((128, H), jnp.float32)] * 2 + [
            pltpu.VMEM_SHARED((NACC, H), jnp.float32),
        ] + [pltpu.SemaphoreType.DMA] * 2,
    )
    def k(ta_h, tb_h, srcL_h, srcR_h, dst_h, zero_h,
          outaL_h, outaR_h, outbL_h, outbR_h,
          src_v, dst_v, b0, b1, acc_sh, g0, g1):
        bufs = (b0, b1)
        gs = (g0, g1)
        ss = ()
        cid = lax.axis_index("c")
        sid = lax.axis_index("s")
        r0 = sid * RPT_ACC

        @pl.when(cid == 0)
        def _():
            pltpu.sync_copy(srcL_h.at[sid], src_v)

        @pl.when(cid == 1)
        def _():
            pltpu.sync_copy(srcR_h.at[sid], src_v)

        pltpu.sync_copy(dst_h.at[sid], dst_v)

        pltpu.sync_copy(zero_h.at[pl.ds(r0, RPT_ACC)],
                        acc_sh.at[pl.ds(r0, RPT_ACC)])
        plsc.subcore_barrier()
        _chunk_loop(ta_h, src_v, dst_v, acc_sh, bufs, gs, ss)
        plsc.subcore_barrier()

        @pl.when(cid == 0)
        def _():
            pltpu.sync_copy(acc_sh.at[pl.ds(r0, RPT_ACC)],
                            outaL_h.at[pl.ds(r0, RPT_ACC)])

        @pl.when(cid == 1)
        def _():
            pltpu.sync_copy(acc_sh.at[pl.ds(r0, RPT_ACC)],
                            outaR_h.at[pl.ds(r0, RPT_ACC)])

        plsc.subcore_barrier()
        pltpu.sync_copy(zero_h.at[pl.ds(r0, RPT_ACC)],
                        acc_sh.at[pl.ds(r0, RPT_ACC)])
        plsc.subcore_barrier()
        _chunk_loop(tb_h, src_v, dst_v, acc_sh, bufs, gs, ss)
        plsc.subcore_barrier()

        @pl.when(cid == 0)
        def _():
            pltpu.sync_copy(acc_sh.at[pl.ds(r0, RPT_ACC)],
                            outbL_h.at[pl.ds(r0, RPT_ACC)])

        @pl.when(cid == 1)
        def _():
            pltpu.sync_copy(acc_sh.at[pl.ds(r0, RPT_ACC)],
                            outbR_h.at[pl.ds(r0, RPT_ACC)])

    return k(tab2a, tab2b, srcL3, srcR3, dst3, zeros)


_BLK = 1024
_GRID = NP // _BLK


def _ln(h, g, b):
    mu = jnp.mean(h, axis=-1, keepdims=True)
    v = jnp.mean((h - mu) * (h - mu), axis=-1, keepdims=True)
    return (h - mu) * lax.rsqrt(v + EPS) * g + b


def _row_spec(w):
    return pl.BlockSpec((_BLK, w), lambda i: (i, 0))


def _full_spec(shape):
    nd = len(shape)
    return pl.BlockSpec(shape, lambda i: (0,) * nd)


def _tc_round1(xp, accL, accR, degp, wlin, ws1, wn1, bpack, lnpack):
    def body(x_r, al_r, ar_r, dp_r, wl_r, ws_r, wn_r, bp_r, ln_r,
             out1_r, z2_r, z3_r, inv_r):
        al = al_r[...]
        ar = ar_r[...]
        deg = dp_r[...][:, 0:1]
        pos = deg > 0.0
        inv = jnp.where(pos, 1.0 / jnp.maximum(deg, 1.0), 0.0)
        mcol = jnp.where(pos, 1.0, 0.0)
        accx = jnp.concatenate([al, ar], axis=1)
        ax = jnp.where(pos, accx * inv, 0.0)
        xb = x_r[...]
        wl = wl_r[...]
        ws = ws_r[...]
        wn = wn_r[...]
        bp = bp_r[...]
        lnp = ln_r[...]
        outs = []
        for l in range(3):
            linb = bp[l:l + 1, :]
            convb = bp[3 + l:4 + l, :]
            h = jnp.dot(xb, wl[l], preferred_element_type=jnp.float32) + linb
            a = jnp.dot(ax, wl[l], preferred_element_type=jnp.float32) + mcol * linb
            pre = (jnp.dot(h, ws[l], preferred_element_type=jnp.float32)
                   + jnp.dot(a, wn[l], preferred_element_type=jnp.float32) + convb)
            outs.append(pre)
        out1_r[...] = outs[0]
        z2_r[...] = jnp.maximum(_ln(outs[1], lnp[0:1, :], lnp[1:2, :]), 0.0)
        z3_r[...] = jnp.maximum(_ln(outs[2], lnp[2:3, :], lnp[3:4, :]), 0.0)
        inv_r[...] = jnp.broadcast_to(inv, (_BLK, D))

    out_shapes = [jax.ShapeDtypeStruct((NP, D), jnp.float32)] * 4
    return pl.pallas_call(
        body,
        grid=(_GRID,),
        in_specs=[
            _row_spec(D), _row_spec(H), _row_spec(H),
            pl.BlockSpec((_BLK, 8), lambda i: (i, 0)),
            _full_spec((3, D, D)), _full_spec((3, D, D)), _full_spec((3, D, D)),
            _full_spec((8, D)), _full_spec((8, D)),
        ],
        out_specs=[_row_spec(D)] * 4,
        out_shape=out_shapes,
    )(xp, accL, accR, degp, wlin, ws1, wn1, bpack, lnpack)


def _tc_round2(z2, z3a, a2L, a2R, a3L, a3R, inv, ws2, wn2, bp2):
    def body(z2_r, z3_r, a2l_r, a2r_r, a3l_r, a3r_r, inv_r, ws_r, wn_r, bp_r,
             out2_r, z3b_r):
        inv = inv_r[...][:, 0:1]
        pos = inv > 0.0
        ws = ws_r[...]
        wn = wn_r[...]
        bp = bp_r[...]
        a2 = jnp.concatenate([a2l_r[...], a2r_r[...]], axis=1)
        a3 = jnp.concatenate([a3l_r[...], a3r_r[...]], axis=1)
        a2 = jnp.where(pos, a2 * inv, 0.0)
        a3 = jnp.where(pos, a3 * inv, 0.0)
        out2_r[...] = (jnp.dot(z2_r[...], ws[0], preferred_element_type=jnp.float32)
                       + jnp.dot(a2, wn[0], preferred_element_type=jnp.float32)
                       + bp[0:1, :])
        pre = (jnp.dot(z3_r[...], ws[1], preferred_element_type=jnp.float32)
               + jnp.dot(a3, wn[1], preferred_element_type=jnp.float32)
               + bp[1:2, :])
        z3b_r[...] = jnp.maximum(_ln(pre, bp[2:3, :], bp[3:4, :]), 0.0)

    out_shapes = [jax.ShapeDtypeStruct((NP, D), jnp.float32)] * 2
    return pl.pallas_call(
        body,
        grid=(_GRID,),
        in_specs=[
            _row_spec(D), _row_spec(D),
            _row_spec(H), _row_spec(H), _row_spec(H), _row_spec(H),
            _row_spec(D),
            _full_spec((2, D, D)), _full_spec((2, D, D)), _full_spec((8, D)),
        ],
        out_specs=[_row_spec(D)] * 2,
        out_shape=out_shapes,
    )(z2, z3a, a2L, a2R, a3L, a3R, inv, ws2, wn2, bp2)


def _tc_round3(out1, out2, z3b, a4L, a4R, inv, batch2d, ws3, wn3, wf, fp):
    def body(o1_r, o2_r, z3_r, a4l_r, a4r_r, inv_r, b_r, ws_r, wn_r, wf_r, fp_r,
             node_r, graph_r, gsum):
        i = pl.program_id(0)
        inv = inv_r[...][:, 0:1]
        pos = inv > 0.0
        fp = fp_r[...]
        wf = wf_r[...]
        a4 = jnp.concatenate([a4l_r[...], a4r_r[...]], axis=1)
        a4 = jnp.where(pos, a4 * inv, 0.0)
        out3 = (jnp.dot(z3_r[...], ws_r[...], preferred_element_type=jnp.float32)
                + jnp.dot(a4, wn_r[...], preferred_element_type=jnp.float32)
                + fp[3:4, :])
        xf = o1_r[...] + o2_r[...] + out3
        node_r[...] = jnp.maximum(
            _ln(jnp.dot(xf, wf, preferred_element_type=jnp.float32) + fp[0:1, :],
                fp[1:2, :], fp[2:3, :]), 0.0)

        @pl.when(i == 0)
        def _():
            gsum[...] = jnp.zeros((G, D), jnp.float32)

        ids = b_r[...]
        iota = lax.broadcasted_iota(jnp.int32, (_BLK, G), 1)
        oh = (ids == iota).astype(jnp.float32)
        gsum[...] += lax.dot_general(oh, xf, (((0,), (0,)), ((), ())),
                                     preferred_element_type=jnp.float32)

        @pl.when(i == _GRID - 1)
        def _():
            gs = gsum[...]
            graph_r[...] = jnp.maximum(
                _ln(jnp.dot(gs, wf, preferred_element_type=jnp.float32) + fp[0:1, :],
                    fp[1:2, :], fp[2:3, :]), 0.0)

    out_shapes = [jax.ShapeDtypeStruct((NP, D), jnp.float32),
                  jax.ShapeDtypeStruct((G, D), jnp.float32)]
    return pl.pallas_call(
        body,
        grid=(_GRID,),
        in_specs=[
            _row_spec(D), _row_spec(D), _row_spec(D),
            _row_spec(H), _row_spec(H),
            _row_spec(D),
            pl.BlockSpec((_BLK, 1), lambda i: (i, 0)),
            _full_spec((D, D)), _full_spec((D, D)), _full_spec((D, D)),
            _full_spec((8, D)),
        ],
        out_specs=[_row_spec(D), _full_spec((G, D))],
        out_shape=out_shapes,
        scratch_shapes=[pltpu.VMEM((G, D), jnp.float32)],
    )(out1, out2, z3b, a4L, a4R, inv, batch2d, ws3, wn3, wf, fp)


def _edge_layout(idx, fill):
    a = idx.reshape(NS, EPT)
    a = jnp.pad(a, ((0, 0), (0, CH * 128 - EPT)), constant_values=fill)
    return a.reshape(NS, CH, 128)


def kernel(x, edge_index, batch, params):
    src = edge_index[0].astype(jnp.int32)
    dst = edge_index[1].astype(jnp.int32)

    xp = jnp.pad(x, ((0, NP - N), (0, 0)))

    srcL = _edge_layout(src * 2, 0)
    srcR = _edge_layout(src * 2 + 1, 1)
    dstB = _edge_layout(dst, N)

    z64 = jnp.zeros((NP, H), jnp.float32)

    batch2d = jnp.pad(batch.astype(jnp.int32), (0, NP - N),
                      constant_values=G).reshape(NP, 1)

    lv = params["levels"]
    f = params["fusion"]
    zrow = jnp.zeros((D,), jnp.float32)
    wlin = jnp.stack([l["lin_W"] for l in lv])
    ws1 = jnp.stack([l["convs"][0]["Ws"] for l in lv])
    wn1 = jnp.stack([l["convs"][0]["Wn"] for l in lv])
    bpack = jnp.stack(
        [lv[0]["lin_b"], lv[1]["lin_b"], lv[2]["lin_b"],
         lv[0]["convs"][0]["b"], lv[1]["convs"][0]["b"], lv[2]["convs"][0]["b"],
         zrow, zrow])
    lnpack = jnp.stack(
        [lv[1]["lns"][0]["g"], lv[1]["lns"][0]["b"],
         lv[2]["lns"][0]["g"], lv[2]["lns"][0]["b"], zrow, zrow, zrow, zrow])
    ws2 = jnp.stack([lv[1]["convs"][1]["Ws"], lv[2]["convs"][1]["Ws"]])
    wn2 = jnp.stack([lv[1]["convs"][1]["Wn"], lv[2]["convs"][1]["Wn"]])
    bp2 = jnp.stack(
        [lv[1]["convs"][1]["b"], lv[2]["convs"][1]["b"],
         lv[2]["lns"][1]["g"], lv[2]["lns"][1]["b"], zrow, zrow, zrow, zrow])
    ws3 = lv[2]["convs"][2]["Ws"]
    wn3 = lv[2]["convs"][2]["Wn"]
    fp = jnp.stack(
        [f["b"], f["ln_g"], f["ln_b"], lv[2]["convs"][2]["b"],
         zrow, zrow, zrow, zrow])

    accL, accR, degp = _agg_one(xp.reshape(NP * 2, H), srcL, srcR, dstB, z64,
                                with_deg=True)

    out1, z2, z3a, inv = _tc_round1(xp, accL, accR, degp,
                                    wlin, ws1, wn1, bpack, lnpack)

    a2L, a2R, a3L, a3R = _agg_two(z2.reshape(NP * 2, H), z3a.reshape(NP * 2, H),
                                  srcL, srcR, dstB, z64)

    out2, z3b = _tc_round2(z2, z3a, a2L, a2R, a3L, a3R, inv, ws2, wn2, bp2)

    a4L, a4R = _agg_one(z3b.reshape(NP * 2, H), srcL, srcR, dstB, z64,
                        with_deg=False)

    node_p, graph_out = _tc_round3(out1, out2, z3b, a4L, a4R, inv,
                                   batch2d, ws3, wn3, f["W"], fp)

    return (node_p[:N], graph_out)
